# baseline (speedup 1.0000x reference)
import jax
import jax.numpy as jnp
from jax import lax
from jax.experimental import pallas as pl
from jax.experimental.pallas import tpu as pltpu
from jax.experimental.pallas import tpu_sc as plsc

N = 10000
D = 128
E = 106667
NPAD = 10112
DUMP = N
NSC = 2
NSUB = 16
NW = NSC * NSUB
CH = 112
NCH0 = 22
NCH1 = 38
NCHM = max(NCH0, NCH1)
EPAD = (NCH0 + NCH1) * NSUB * CH
RPS = NPAD // NSUB


def _sc_agg_build(with_deg: bool):
  mesh = plsc.VectorSubcoreMesh(core_axis_name="c", subcore_axis_name="s")

  out_type = [jax.ShapeDtypeStruct((6, NPAD, D), jnp.float32)]
  scratch = [
      pltpu.VMEM((NCHM, CH), jnp.int32),
      pltpu.VMEM((NCHM, CH), jnp.int32),
      pltpu.VMEM((CH, D), jnp.float32),
      pltpu.VMEM((CH, D), jnp.float32),
      pltpu.VMEM_SHARED((NPAD, D), jnp.float32),
      pltpu.SemaphoreType.DMA,
      pltpu.SemaphoreType.DMA,
  ]
  if with_deg:
    scratch.append(pltpu.VMEM((NPAD,), jnp.float32))
    out_type.append(jax.ShapeDtypeStruct((3, NW, NPAD), jnp.float32))

  def body(table_hbm, src_hbm, dst_hbm, zrow_hbm, *rest):
    if with_deg:
      (agg_out, deg_out, src_v, dst_v, rows_a, rows_b, agg_sh, sem_a, sem_b,
       deg_v) = rest
    else:
      (agg_out, src_v, dst_v, rows_a, rows_b, agg_sh, sem_a, sem_b) = rest
    c = lax.axis_index("c")
    s = lax.axis_index("s")
    wid = c * NSUB + s
    base = s * RPS
    ones16 = jnp.full((16,), 1.0, dtype=jnp.float32)
    zeros16 = jnp.zeros((16,), dtype=jnp.float32)

    def deg_adds(j):
      for k in range(CH // 16):
        idx = dst_v[j, pl.ds(k * 16, 16)]
        plsc.addupdate_scatter(deg_v, [idx], ones16)

    def edge_loop(nch):
      pltpu.async_copy(table_hbm.at[src_v.at[0]], rows_a, sem_a)

      def pair(p, carry):
        j = 2 * p
        pltpu.make_async_copy(
            table_hbm.at[src_v.at[j]], rows_a, sem_a).wait()
        pltpu.async_copy(table_hbm.at[src_v.at[j + 1]], rows_b, sem_b)
        pltpu.sync_copy(rows_a, agg_sh.at[dst_v.at[j]], add=True)
        if with_deg:
          deg_adds(j)
        pltpu.make_async_copy(
            table_hbm.at[src_v.at[j + 1]], rows_b, sem_b).wait()
        pltpu.async_copy(
            table_hbm.at[src_v.at[jnp.minimum(j + 2, nch - 1)]], rows_a,
            sem_a)
        pltpu.sync_copy(rows_b, agg_sh.at[dst_v.at[j + 1]], add=True)
        if with_deg:
          deg_adds(j + 1)
        return carry

      lax.fori_loop(0, nch // 2, pair, 0)
      pltpu.make_async_copy(
          table_hbm.at[src_v.at[nch - 1]], rows_a, sem_a).wait()

    for r in range(3):
      pltpu.sync_copy(zrow_hbm, agg_sh.at[pl.ds(base, RPS)])
      if with_deg:
        def zstep(i, carry):
          deg_v[pl.ds(i * 16, 16)] = zeros16
          return carry
        lax.fori_loop(0, NPAD // 16, zstep, 0)
      pltpu.sync_copy(src_hbm.at[r, wid], src_v)
      pltpu.sync_copy(dst_hbm.at[r, wid], dst_v)
      plsc.subcore_barrier()

      pl.when(c == 0)(lambda: edge_loop(NCH0))
      pl.when(c != 0)(lambda: edge_loop(NCH1))

      plsc.subcore_barrier()
      oc = r * 2 + c
      pltpu.sync_copy(agg_sh.at[pl.ds(base, RPS)],
                      agg_out.at[oc, pl.ds(base, RPS)])
      if with_deg:
        pltpu.sync_copy(deg_v, deg_out.at[r, wid])

  return pl.kernel(
      body,
      out_type=tuple(out_type),
      mesh=mesh,
      scratch_types=tuple(scratch),
      compiler_params=pltpu.CompilerParams(needs_layout_passes=False),
  )


def _tc_combine_body(agg_ref, deg_ref, w_ref, b_ref, out_ref):
  acc = jnp.zeros(out_ref.shape, dtype=jnp.float32)
  for r in range(3):
    a = agg_ref[2 * r] + agg_ref[2 * r + 1]
    deg = jnp.sum(deg_ref[r], axis=1, keepdims=True)
    inv = 1.0 / jnp.maximum(deg, 1.0)
    acc = acc + jnp.dot(a * inv, w_ref[r], preferred_element_type=jnp.float32)
  acc = acc + jnp.sum(b_ref[...], axis=0)[None, :]
  out_ref[...] = acc


_BN = 1264


def _tc_combine(agg, deg_t, W, b):
  grid = NPAD // _BN
  return pl.pallas_call(
      _tc_combine_body,
      grid=(grid,),
      in_specs=[
          pl.BlockSpec((6, _BN, D), lambda i: (0, i, 0)),
          pl.BlockSpec((3, _BN, NW), lambda i: (0, i, 0)),
          pl.BlockSpec((3, D, D), lambda i: (0, 0, 0)),
          pl.BlockSpec((3, D), lambda i: (0, 0)),
      ],
      out_specs=pl.BlockSpec((_BN, D), lambda i: (i, 0)),
      out_shape=jax.ShapeDtypeStruct((NPAD, D), jnp.float32),
  )(agg, deg_t, W, b)


def _split_indices(vec, fill):
  vec = jnp.concatenate(
      [vec.astype(jnp.int32),
       jnp.full((EPAD - E,), fill, jnp.int32)])
  n0 = NSUB * NCH0 * CH
  c0 = vec[:n0].reshape(NSUB, NCH0, CH)
  c1 = vec[n0:].reshape(NSUB, NCH1, CH)
  c0 = jnp.pad(c0, ((0, 0), (0, NCHM - NCH0), (0, 0)), constant_values=fill)
  c1 = jnp.pad(c1, ((0, 0), (0, NCHM - NCH1), (0, 0)), constant_values=fill)
  return jnp.concatenate([c0, c1], axis=0)


def kernel(x, edge_index_r0, edge_index_r1, edge_index_r2,
           W1_r0, b1_r0, W1_r1, b1_r1, W1_r2, b1_r2,
           W2_r0, b2_r0, W2_r1, b2_r1, W2_r2, b2_r2):
  srcs, dsts = [], []
  for ei in (edge_index_r0, edge_index_r1, edge_index_r2):
    srcs.append(_split_indices(ei[0], 0))
    dsts.append(_split_indices(ei[1], DUMP))
  src_idx = jnp.stack(srcs)
  dst_idx = jnp.stack(dsts)

  zrow = jnp.zeros((RPS, D), jnp.float32)

  sc_l1 = _sc_agg_build(with_deg=True)
  sc_l2 = _sc_agg_build(with_deg=False)

  agg1, deg = sc_l1(x, src_idx, dst_idx, zrow)
  deg_t = jnp.transpose(deg, (0, 2, 1))
  W1 = jnp.stack([W1_r0, W1_r1, W1_r2])
  b1 = jnp.stack([b1_r0, b1_r1, b1_r2])
  h_pad = _tc_combine(agg1, deg_t, W1, b1)

  (agg2,) = sc_l2(h_pad, src_idx, dst_idx, zrow)
  W2 = jnp.stack([W2_r0, W2_r1, W2_r2])
  b2 = jnp.stack([b2_r0, b2_r1, b2_r2])
  out_pad = _tc_combine(agg2, deg_t, W2, b2)
  return out_pad[:N]


# speedup vs baseline: 1.1521x; 1.1521x over previous
import jax
import jax.numpy as jnp
from jax import lax
from jax.experimental import pallas as pl
from jax.experimental.pallas import tpu as pltpu
from jax.experimental.pallas import tpu_sc as plsc

N = 10000
D = 128
E = 106667
NPAD = 10112
DUMP = N
NSC = 2
NSUB = 16
NW = NSC * NSUB
CH = 112
NCH0 = 38
NCH1 = 22
NCHM = max(NCH0, NCH1)
EPAD = (NCH0 + NCH1) * NSUB * CH
RPS = NPAD // NSUB


def _sc_agg_build(with_deg: bool):
  mesh = plsc.VectorSubcoreMesh(core_axis_name="c", subcore_axis_name="s")

  out_type = [jax.ShapeDtypeStruct((6, NPAD, D), jnp.float32)]
  scratch = [
      pltpu.VMEM((NCHM, CH), jnp.int32),
      pltpu.VMEM((NCHM, CH), jnp.int32),
      pltpu.VMEM((CH, D), jnp.float32),
      pltpu.VMEM((CH, D), jnp.float32),
      pltpu.VMEM_SHARED((NPAD, D), jnp.float32),
      pltpu.SemaphoreType.DMA,
      pltpu.SemaphoreType.DMA,
  ]
  if with_deg:
    scratch.append(pltpu.VMEM((NPAD,), jnp.float32))
    out_type.append(jax.ShapeDtypeStruct((3, NW, NPAD), jnp.float32))

  def body(table_hbm, src_hbm, dst_hbm, zrow_hbm, *rest):
    if with_deg:
      (agg_out, deg_out, src_v, dst_v, rows_a, rows_b, agg_sh, sem_a, sem_b,
       deg_v) = rest
    else:
      (agg_out, src_v, dst_v, rows_a, rows_b, agg_sh, sem_a, sem_b) = rest
    c = lax.axis_index("c")
    s = lax.axis_index("s")
    wid = c * NSUB + s
    base = s * RPS
    ones16 = jnp.full((16,), 1.0, dtype=jnp.float32)
    zeros16 = jnp.zeros((16,), dtype=jnp.float32)

    def deg_adds(j):
      for k in range(CH // 16):
        idx = dst_v[j, pl.ds(k * 16, 16)]
        plsc.addupdate_scatter(deg_v, [idx], ones16)

    def edge_loop(nch):
      pltpu.async_copy(table_hbm.at[src_v.at[0]], rows_a, sem_a)

      def pair(p, carry):
        j = 2 * p
        pltpu.make_async_copy(
            table_hbm.at[src_v.at[j]], rows_a, sem_a).wait()
        pltpu.async_copy(table_hbm.at[src_v.at[j + 1]], rows_b, sem_b)
        pltpu.sync_copy(rows_a, agg_sh.at[dst_v.at[j]], add=True)
        if with_deg:
          deg_adds(j)
        pltpu.make_async_copy(
            table_hbm.at[src_v.at[j + 1]], rows_b, sem_b).wait()
        pltpu.async_copy(
            table_hbm.at[src_v.at[jnp.minimum(j + 2, nch - 1)]], rows_a,
            sem_a)
        pltpu.sync_copy(rows_b, agg_sh.at[dst_v.at[j + 1]], add=True)
        if with_deg:
          deg_adds(j + 1)
        return carry

      lax.fori_loop(0, nch // 2, pair, 0)
      pltpu.make_async_copy(
          table_hbm.at[src_v.at[nch - 1]], rows_a, sem_a).wait()

    for r in range(3):
      pltpu.sync_copy(zrow_hbm, agg_sh.at[pl.ds(base, RPS)])
      if with_deg:
        def zstep(i, carry):
          deg_v[pl.ds(i * 16, 16)] = zeros16
          return carry
        lax.fori_loop(0, NPAD // 16, zstep, 0)
      pltpu.sync_copy(src_hbm.at[r, wid], src_v)
      pltpu.sync_copy(dst_hbm.at[r, wid], dst_v)
      plsc.subcore_barrier()

      pl.when(c == 0)(lambda: edge_loop(NCH0))
      pl.when(c != 0)(lambda: edge_loop(NCH1))

      plsc.subcore_barrier()
      oc = r * 2 + c
      pltpu.sync_copy(agg_sh.at[pl.ds(base, RPS)],
                      agg_out.at[oc, pl.ds(base, RPS)])
      if with_deg:
        pltpu.sync_copy(deg_v, deg_out.at[r, wid])

  return pl.kernel(
      body,
      out_type=tuple(out_type),
      mesh=mesh,
      scratch_types=tuple(scratch),
      compiler_params=pltpu.CompilerParams(needs_layout_passes=False),
  )


def _tc_combine_body(agg_ref, deg_ref, w_ref, b_ref, out_ref):
  acc = jnp.zeros(out_ref.shape, dtype=jnp.float32)
  for r in range(3):
    a = agg_ref[2 * r] + agg_ref[2 * r + 1]
    deg = jnp.sum(deg_ref[r], axis=1, keepdims=True)
    inv = 1.0 / jnp.maximum(deg, 1.0)
    acc = acc + jnp.dot(a * inv, w_ref[r], preferred_element_type=jnp.float32)
  acc = acc + jnp.sum(b_ref[...], axis=0)[None, :]
  out_ref[...] = acc


_BN = 1264


def _tc_combine(agg, deg_t, W, b):
  grid = NPAD // _BN
  return pl.pallas_call(
      _tc_combine_body,
      grid=(grid,),
      in_specs=[
          pl.BlockSpec((6, _BN, D), lambda i: (0, i, 0)),
          pl.BlockSpec((3, _BN, NW), lambda i: (0, i, 0)),
          pl.BlockSpec((3, D, D), lambda i: (0, 0, 0)),
          pl.BlockSpec((3, D), lambda i: (0, 0)),
      ],
      out_specs=pl.BlockSpec((_BN, D), lambda i: (i, 0)),
      out_shape=jax.ShapeDtypeStruct((NPAD, D), jnp.float32),
  )(agg, deg_t, W, b)


def _split_indices(vec, fill):
  vec = jnp.concatenate(
      [vec.astype(jnp.int32),
       jnp.full((EPAD - E,), fill, jnp.int32)])
  n0 = NSUB * NCH0 * CH
  c0 = vec[:n0].reshape(NSUB, NCH0, CH)
  c1 = vec[n0:].reshape(NSUB, NCH1, CH)
  c0 = jnp.pad(c0, ((0, 0), (0, NCHM - NCH0), (0, 0)), constant_values=fill)
  c1 = jnp.pad(c1, ((0, 0), (0, NCHM - NCH1), (0, 0)), constant_values=fill)
  return jnp.concatenate([c0, c1], axis=0)


def kernel(x, edge_index_r0, edge_index_r1, edge_index_r2,
           W1_r0, b1_r0, W1_r1, b1_r1, W1_r2, b1_r2,
           W2_r0, b2_r0, W2_r1, b2_r1, W2_r2, b2_r2):
  srcs, dsts = [], []
  for ei in (edge_index_r0, edge_index_r1, edge_index_r2):
    srcs.append(_split_indices(ei[0], 0))
    dsts.append(_split_indices(ei[1], DUMP))
  src_idx = jnp.stack(srcs)
  dst_idx = jnp.stack(dsts)

  zrow = jnp.zeros((RPS, D), jnp.float32)

  sc_l1 = _sc_agg_build(with_deg=True)
  sc_l2 = _sc_agg_build(with_deg=False)

  agg1, deg = sc_l1(x, src_idx, dst_idx, zrow)
  deg_t = jnp.transpose(deg, (0, 2, 1))
  W1 = jnp.stack([W1_r0, W1_r1, W1_r2])
  b1 = jnp.stack([b1_r0, b1_r1, b1_r2])
  h_pad = _tc_combine(agg1, deg_t, W1, b1)

  (agg2,) = sc_l2(h_pad, src_idx, dst_idx, zrow)
  W2 = jnp.stack([W2_r0, W2_r1, W2_r2])
  b2 = jnp.stack([b2_r0, b2_r1, b2_r2])
  out_pad = _tc_combine(agg2, deg_t, W2, b2)
  return out_pad[:N]
